# baseline (speedup 1.0000x reference)
import functools

import jax
import jax.numpy as jnp
from jax import lax
from jax.experimental import pallas as pl
from jax.experimental.pallas import tpu as pltpu
from jax.experimental.pallas import tpu_sc as plsc

K_NN = 8
RADIUS = 2.5
_BIGQ = 1e30
_NLANE = 16


def _sel_body(cx_hbm, cy_hbm, h_hbm, m_hbm, out_hbm,
              cxv, cyv, hv, mv, dref, wscr):
    N = cxv.shape[0]
    wid = lax.axis_index("s") * 2 + lax.axis_index("c")
    pltpu.sync_copy(cx_hbm.at[wid], cxv)
    pltpu.sync_copy(cy_hbm.at[wid], cyv)
    pltpu.sync_copy(h_hbm.at[wid], hv)
    pltpu.sync_copy(m_hbm.at[wid], mv)

    def row_body(i, _):
        cxi = cxv[i]
        cyi = cyv[i]
        hi = hv[i]
        mi = mv[i]
        rad2 = hi * hi * (RADIUS * RADIUS)

        for j in range(N):
            dx = cxi - cxv[j]
            dy = cyi - cyv[j]
            q = dx * dx + dy * dy + 1e-6
            dref[j] = q + (1.0 - mi * mv[j]) * _BIGQ
        dref[i] = jnp.full((_NLANE,), _BIGQ, jnp.float32)

        dks = [dref[k] for k in range(N)]
        for j in range(N):
            dj = dks[j]
            rank = jnp.zeros((_NLANE,), jnp.float32)
            for k in range(N):
                c = (dks[k] <= dj) if k < j else (dks[k] < dj)
                rank = rank + jnp.where(c, 1.0, 0.0)
            sel = (rank < float(K_NN)) & (dj < rad2)
            wscr[j, i] = jnp.where(sel, 1.0, 0.0)
        return 0

    lax.fori_loop(0, N, row_body, 0)
    pltpu.sync_copy(wscr, out_hbm.at[wid])


def _select_sc(cx, cy, h, m, interpret=False):
    BT, N = cx.shape
    NCHUNK = BT // _NLANE

    def chunked(a):
        return a.reshape(NCHUNK, _NLANE, N).transpose(0, 2, 1)

    mesh = plsc.VectorSubcoreMesh(core_axis_name="c", subcore_axis_name="s",
                                  num_cores=2, num_subcores=16)
    f32 = jnp.float32
    sck = pl.kernel(
        _sel_body,
        out_type=jax.ShapeDtypeStruct((NCHUNK, N, N, _NLANE), f32),
        mesh=mesh,
        scratch_types=[pltpu.VMEM((N, _NLANE), f32)] * 5
        + [pltpu.VMEM((N, N, _NLANE), f32)],
        interpret=interpret,
    )
    w = sck(chunked(cx), chunked(cy), chunked(h), chunked(m))
    return w.transpose(0, 3, 1, 2).reshape(BT, N, N)


def _graph_body(cxr, cyr, mr, cxc, cyc, hc, wsel_ref, x_ref,
                w1ea, w1eb, wc, b1e, w2e, b2e,
                w1na, w1nb, b1n, w2n, b2n, gam, bet,
                out_ref):
    G, N, D = x_ref.shape
    H = w1ea.shape[1]

    dxT = cxc[...] - cxr[...]
    dyT = cyc[...] - cyr[...]
    distT = jnp.sqrt(dxT * dxT + dyT * dyT + 1e-6)
    hcv = hc[...]
    dxnT = dxT / hcv
    dynT = dyT / hcv
    distnT = distT / hcv

    wT = wsel_ref[...]

    x2 = x_ref[...].reshape(G * N, D)
    a = jnp.dot(x2, w1ea[...], preferred_element_type=jnp.float32)
    b = jnp.dot(x2, w1eb[...], preferred_element_type=jnp.float32)
    a3 = a.reshape(G, N, H)
    b3 = b.reshape(G, N, H) + b1e[...]
    wc0 = wc[0:1, :].reshape(1, 1, H)
    wc1 = wc[1:2, :].reshape(1, 1, H)
    wc2 = wc[2:3, :].reshape(1, 1, H)

    s_parts = []
    c_parts = []
    for i in range(N):
        ai = a3[:, i:i + 1, :]
        ei = (dxnT[:, :, i:i + 1] * wc0 + dynT[:, :, i:i + 1] * wc1
              + distnT[:, :, i:i + 1] * wc2)
        h1 = jnp.maximum(ai + b3 + ei, 0.0)
        wi = wT[:, :, i:i + 1]
        s_parts.append(jnp.sum(wi * h1, axis=1, keepdims=True))
        c_parts.append(jnp.sum(wi, axis=1, keepdims=True))
    s = jnp.concatenate(s_parts, axis=1).reshape(G * N, H)
    cnt = jnp.concatenate(c_parts, axis=1).reshape(G * N, 1)

    denom = jnp.maximum(cnt, 1.0)
    hasn = (cnt > 0.0).astype(jnp.float32)
    agg = jnp.dot(s, w2e[...], preferred_element_type=jnp.float32) / denom \
        + b2e[...] * hasn
    n1 = jnp.maximum(
        jnp.dot(x2, w1na[...], preferred_element_type=jnp.float32)
        + jnp.dot(agg, w1nb[...], preferred_element_type=jnp.float32)
        + b1n[...], 0.0)
    delta = (jnp.dot(n1, w2n[...], preferred_element_type=jnp.float32)
             + b2n[...]) * hasn
    y = x2 + delta
    mu = jnp.mean(y, axis=1, keepdims=True)
    yc = y - mu
    var = jnp.mean(yc * yc, axis=1, keepdims=True)
    out = yc / jnp.sqrt(var + 1e-5) * gam[...] + bet[...]
    out = out * mr[...].reshape(G * N, 1)
    out_ref[...] = out.reshape(G, N, D)


@functools.partial(jax.jit, static_argnames=("interpret",))
def kernel(emb, bboxes, person_mask, W1e, b1e, W2e, b2e, W1n, b1n, W2n, b2n,
           gamma, beta, interpret=False):
    B, T, N, D = emb.shape
    BT = B * T
    H = W1e.shape[1]
    G = 64
    x = emb.reshape(BT, N, D)
    boxes = bboxes.reshape(BT, N, 4)
    cx = boxes[:, :, 0]
    cy = boxes[:, :, 1]
    h = jnp.maximum(boxes[:, :, 3], 1e-6)
    m = person_mask.reshape(BT, N).astype(jnp.float32)

    wselT = _select_sc(cx, cy, h, m, interpret=interpret)

    cxr, cyr, mr = cx[:, :, None], cy[:, :, None], m[:, :, None]
    cxc, cyc, hc = cx[:, None, :], cy[:, None, :], h[:, None, :]

    row = pl.BlockSpec((G, N, 1), lambda g: (g, 0, 0))
    col = pl.BlockSpec((G, 1, N), lambda g: (g, 0, 0))
    pair = pl.BlockSpec((G, N, N), lambda g: (g, 0, 0))
    xsp = pl.BlockSpec((G, N, D), lambda g: (g, 0, 0))

    def full(arr):
        return pl.BlockSpec(arr.shape, lambda g: (0,) * arr.ndim)

    w1ea, w1eb, wce = W1e[:D], W1e[D:2 * D], W1e[2 * D:]
    w1na, w1nb = W1n[:D], W1n[D:]
    wts = (w1ea, w1eb, wce, b1e.reshape(1, H), W2e, b2e.reshape(1, D),
           w1na, w1nb, b1n.reshape(1, H), W2n, b2n.reshape(1, D),
           gamma.reshape(1, D), beta.reshape(1, D))

    out = pl.pallas_call(
        _graph_body,
        grid=(BT // G,),
        in_specs=[row, row, row, col, col, col, pair, xsp]
        + [full(w) for w in wts],
        out_specs=xsp,
        out_shape=jax.ShapeDtypeStruct((BT, N, D), jnp.float32),
        interpret=interpret,
    )(cxr, cyr, mr, cxc, cyc, hc, wselT, x, *wts)
    return out.reshape(B, T, N, D)


# speedup vs baseline: 1.2094x; 1.2094x over previous
import functools

import jax
import jax.numpy as jnp
from jax import lax
from jax.experimental import pallas as pl
from jax.experimental.pallas import tpu as pltpu
from jax.experimental.pallas import tpu_sc as plsc

K_NN = 8
RADIUS = 2.5
_BIGQ = 1e30
_PEN = 1e9
_NLANE = 16


def _sel_body(cx_hbm, cy_hbm, h_hbm, m_hbm, out_hbm, cnt_hbm,
              cxv, cyv, hv, mv, dref, wscr, cscr):
    N = cxv.shape[0]
    wid = lax.axis_index("s") * 2 + lax.axis_index("c")
    pltpu.sync_copy(cx_hbm.at[wid], cxv)
    pltpu.sync_copy(cy_hbm.at[wid], cyv)
    pltpu.sync_copy(h_hbm.at[wid], hv)
    pltpu.sync_copy(m_hbm.at[wid], mv)

    def row_body(i, _):
        cxi = cxv[i]
        cyi = cyv[i]
        hi = hv[i]
        mi = mv[i]
        rad2 = hi * hi * (RADIUS * RADIUS)

        for j in range(N):
            dx = cxi - cxv[j]
            dy = cyi - cyv[j]
            q = dx * dx + dy * dy + 1e-6
            dref[j] = q + (1.0 - mi * mv[j]) * _BIGQ
        dref[i] = jnp.full((_NLANE,), _BIGQ, jnp.float32)

        dks = [dref[k] for k in range(N)]
        cnt = jnp.zeros((_NLANE,), jnp.float32)
        for j in range(N):
            dj = dks[j]
            rank = jnp.zeros((_NLANE,), jnp.float32)
            for k in range(N):
                c = (dks[k] <= dj) if k < j else (dks[k] < dj)
                rank = rank + jnp.where(c, 1.0, 0.0)
            sel = (rank < float(K_NN)) & (dj < rad2)
            cnt = cnt + jnp.where(sel, 1.0, 0.0)
            wscr[j, i] = jnp.where(sel, 0.0, -_PEN)
        cscr[i] = cnt
        return 0

    lax.fori_loop(0, N, row_body, 0)
    pltpu.sync_copy(wscr, out_hbm.at[wid])
    pltpu.sync_copy(cscr, cnt_hbm.at[wid])


def _select_sc(cx, cy, h, m, interpret=False):
    BT, N = cx.shape
    NCHUNK = BT // _NLANE

    def chunked(a):
        return a.reshape(NCHUNK, _NLANE, N).transpose(0, 2, 1)

    mesh = plsc.VectorSubcoreMesh(core_axis_name="c", subcore_axis_name="s",
                                  num_cores=2, num_subcores=16)
    f32 = jnp.float32
    sck = pl.kernel(
        _sel_body,
        out_type=(jax.ShapeDtypeStruct((NCHUNK, N, N, _NLANE), f32),
                  jax.ShapeDtypeStruct((NCHUNK, N, _NLANE), f32)),
        mesh=mesh,
        scratch_types=[pltpu.VMEM((N, _NLANE), f32)] * 4
        + [pltpu.VMEM((N, _NLANE), f32),
           pltpu.VMEM((N, N, _NLANE), f32),
           pltpu.VMEM((N, _NLANE), f32)],
        interpret=interpret,
    )
    pen, cnt = sck(chunked(cx), chunked(cy), chunked(h), chunked(m))
    penT = pen.transpose(0, 3, 1, 2).reshape(BT, N, N)
    cnt3 = cnt.transpose(0, 2, 1).reshape(BT, N, 1)
    return penT, cnt3


def _graph_body(cxr, cyr, mr, cxc, cyc, hc, pen_ref, cnt_ref, x_ref,
                w1ea, w1eb, w4d, b1e, w2e, b2e,
                w1na, w1nb, b1n, w2n, b2n, gam, bet,
                out_ref):
    G, N, D = x_ref.shape
    H = w1ea.shape[1]

    dxT = cxc[...] - cxr[...]
    dyT = cyc[...] - cyr[...]
    distT = jnp.sqrt(dxT * dxT + dyT * dyT + 1e-6)
    hcv = hc[...]
    dxnT = dxT / hcv
    dynT = dyT / hcv
    distnT = distT / hcv

    penT = pen_ref[...]

    x2 = x_ref[...].reshape(G * N, D)
    a = jnp.dot(x2, w1ea[...], preferred_element_type=jnp.float32)
    b = jnp.dot(x2, w1eb[...], preferred_element_type=jnp.float32)
    a3 = a.reshape(G, N, H)
    b3 = b.reshape(G, N, H) + b1e[...]
    b3d = jnp.concatenate([b3, b3], axis=2)

    NH = N // 2
    s_parts = []
    for i in range(NH):
        i2 = i + NH
        fi = jnp.concatenate(
            [dxnT[:, :, i:i + 1], dynT[:, :, i:i + 1],
             distnT[:, :, i:i + 1], penT[:, :, i:i + 1],
             dxnT[:, :, i2:i2 + 1], dynT[:, :, i2:i2 + 1],
             distnT[:, :, i2:i2 + 1], penT[:, :, i2:i2 + 1]], axis=2)
        ei = jnp.dot(fi.reshape(G * N, 8), w4d[...],
                     preferred_element_type=jnp.float32).reshape(G, N, 2 * H)
        aip = jnp.concatenate([a3[:, i:i + 1, :], a3[:, i2:i2 + 1, :]], axis=2)
        h1 = jnp.maximum(aip + b3d + ei, 0.0)
        s_parts.append(jnp.sum(h1, axis=1, keepdims=True))
    s3 = jnp.concatenate(s_parts, axis=1)
    s = jnp.concatenate([s3[:, :, :H], s3[:, :, H:]], axis=1).reshape(G * N, H)
    cnt = cnt_ref[...].reshape(G * N, 1)

    denom = jnp.maximum(cnt, 1.0)
    hasn = (cnt > 0.0).astype(jnp.float32)
    agg = jnp.dot(s, w2e[...], preferred_element_type=jnp.float32) / denom \
        + b2e[...] * hasn
    n1 = jnp.maximum(
        jnp.dot(x2, w1na[...], preferred_element_type=jnp.float32)
        + jnp.dot(agg, w1nb[...], preferred_element_type=jnp.float32)
        + b1n[...], 0.0)
    delta = (jnp.dot(n1, w2n[...], preferred_element_type=jnp.float32)
             + b2n[...]) * hasn
    y = x2 + delta
    mu = jnp.mean(y, axis=1, keepdims=True)
    yc = y - mu
    var = jnp.mean(yc * yc, axis=1, keepdims=True)
    out = yc / jnp.sqrt(var + 1e-5) * gam[...] + bet[...]
    out = out * mr[...].reshape(G * N, 1)
    out_ref[...] = out.reshape(G, N, D)


@functools.partial(jax.jit, static_argnames=("interpret",))
def kernel(emb, bboxes, person_mask, W1e, b1e, W2e, b2e, W1n, b1n, W2n, b2n,
           gamma, beta, interpret=False):
    B, T, N, D = emb.shape
    BT = B * T
    H = W1e.shape[1]
    G = 64
    x = emb.reshape(BT, N, D)
    boxes = bboxes.reshape(BT, N, 4)
    cx = boxes[:, :, 0]
    cy = boxes[:, :, 1]
    h = jnp.maximum(boxes[:, :, 3], 1e-6)
    m = person_mask.reshape(BT, N).astype(jnp.float32)

    penT, cnt3 = _select_sc(cx, cy, h, m, interpret=interpret)

    cxr, cyr, mr = cx[:, :, None], cy[:, :, None], m[:, :, None]
    cxc, cyc, hc = cx[:, None, :], cy[:, None, :], h[:, None, :]

    row = pl.BlockSpec((G, N, 1), lambda g: (g, 0, 0))
    col = pl.BlockSpec((G, 1, N), lambda g: (g, 0, 0))
    pair = pl.BlockSpec((G, N, N), lambda g: (g, 0, 0))
    xsp = pl.BlockSpec((G, N, D), lambda g: (g, 0, 0))

    def full(arr):
        return pl.BlockSpec(arr.shape, lambda g: (0,) * arr.ndim)

    w1ea, w1eb, wce = W1e[:D], W1e[D:2 * D], W1e[2 * D:]
    w1na, w1nb = W1n[:D], W1n[D:]
    w4d = jnp.zeros((8, 2 * H), jnp.float32)
    w4d = w4d.at[0:3, 0:H].set(wce)
    w4d = w4d.at[3, 0:H].set(1.0)
    w4d = w4d.at[4:7, H:2 * H].set(wce)
    w4d = w4d.at[7, H:2 * H].set(1.0)
    wts = (w1ea, w1eb, w4d, b1e.reshape(1, H), W2e, b2e.reshape(1, D),
           w1na, w1nb, b1n.reshape(1, H), W2n, b2n.reshape(1, D),
           gamma.reshape(1, D), beta.reshape(1, D))

    out = pl.pallas_call(
        _graph_body,
        grid=(BT // G,),
        in_specs=[row, row, row, col, col, col, pair, row, xsp]
        + [full(w) for w in wts],
        out_specs=xsp,
        out_shape=jax.ShapeDtypeStruct((BT, N, D), jnp.float32),
        interpret=interpret,
    )(cxr, cyr, mr, cxc, cyc, hc, penT, cnt3, x, *wts)
    return out.reshape(B, T, N, D)
